# baseline (speedup 1.0000x reference)
import dataclasses
import functools

import jax
import jax.numpy as jnp
from jax import lax
from jax.experimental import pallas as pl
from jax.experimental.pallas import tpu as pltpu
from jax.experimental.pallas import tpu_sc as plsc

B, T, V, C = 16384, 200, 10, 3
N_IDX = B * T
L = 16
WIN = 2048
N_WIN = N_IDX // WIN
GRPS = WIN // L


def kernel(idx, weight):
    idx2d = idx.reshape(N_WIN, WIN)
    mesh = plsc.VectorSubcoreMesh(core_axis_name="c", subcore_axis_name="s")

    @functools.partial(
        pl.kernel,
        out_type=jax.ShapeDtypeStruct((N_WIN, WIN * V), jnp.float32),
        mesh=mesh,
        scratch_types=[pltpu.VMEM((128,), jnp.float32),
                       pltpu.VMEM((V, C), jnp.float32)],
        compiler_params=dataclasses.replace(
            pltpu.CompilerParams(), needs_layout_passes=False),
    )
    def sc_kern(idx_hbm, w_hbm, out_hbm, table_vmem, w_vmem):
        pltpu.sync_copy(w_hbm, w_vmem)
        for v in range(7):
            e = lax.iota(jnp.int32, L) + (16 * v)
            k = jnp.minimum(e // V, V - 1)
            j2 = jnp.minimum(e - (e // V) * V, V - 1)
            acc = jnp.zeros((L,), jnp.float32)
            for c in range(C):
                cc = jnp.full((L,), c, jnp.int32)
                acc = acc + (plsc.load_gather(w_vmem, [k, cc]) *
                             plsc.load_gather(w_vmem, [j2, cc]))
            table_vmem[pl.ds(16 * v, L)] = acc

        sidx_pat = lax.iota(jnp.int32, L) * V

        def body(idx_vmem, out_vmem):
            idx_row = idx_vmem.at[0]
            out_row = out_vmem.at[0]

            @pl.loop(0, GRPS)
            def _(g):
                idxv = idx_row[pl.ds(g * L, L)]
                wbase = idxv * V
                sbase = sidx_pat + g * (L * V)
                for j in range(V):
                    vals = plsc.load_gather(table_vmem, [wbase + j])
                    plsc.store_scatter(out_row, [sbase + j], vals)

        pltpu.emit_pipeline(
            body,
            grid=(N_WIN,),
            in_specs=[pl.BlockSpec((1, WIN), index_map=lambda i: (i, 0))],
            out_specs=[pl.BlockSpec((1, WIN * V), index_map=lambda i: (i, 0))],
            core_axis_name=("c", "s"),
            dimension_semantics=(pltpu.PARALLEL,),
        )(idx_hbm, out_hbm)

    out2d = sc_kern(idx2d, weight)
    return out2d.reshape(B, T, V)


# speedup vs baseline: 9.7571x; 9.7571x over previous
import dataclasses
import functools

import jax
import jax.numpy as jnp
from jax import lax
from jax.experimental import pallas as pl
from jax.experimental.pallas import tpu as pltpu
from jax.experimental.pallas import tpu_sc as plsc

B, T, V, C = 16384, 200, 10, 3
L = 16
TH = T // 8
BHQ = B // 128
BH = 4


def kernel(idx, weight):
    idx4 = idx.T.reshape(TH, 8, BHQ, 128).transpose(0, 2, 1, 3)
    mesh = plsc.VectorSubcoreMesh(core_axis_name="c", subcore_axis_name="s")

    @functools.partial(
        pl.kernel,
        out_type=jax.ShapeDtypeStruct((V, TH, BHQ, 8, 128), jnp.float32),
        mesh=mesh,
        scratch_types=[pltpu.VMEM((128,), jnp.float32),
                       pltpu.VMEM((V, C), jnp.float32)],
        compiler_params=dataclasses.replace(
            pltpu.CompilerParams(), needs_layout_passes=False),
    )
    def sc_kern(idx_hbm, w_hbm, out_hbm, table_vmem, w_vmem):
        pltpu.sync_copy(w_hbm, w_vmem)
        for v in range(7):
            e = lax.iota(jnp.int32, L) + (16 * v)
            k = jnp.minimum(e // V, V - 1)
            j2 = jnp.minimum(e - (e // V) * V, V - 1)
            acc = jnp.zeros((L,), jnp.float32)
            for c in range(C):
                cc = jnp.full((L,), c, jnp.int32)
                acc = acc + (plsc.load_gather(w_vmem, [k, cc]) *
                             plsc.load_gather(w_vmem, [j2, cc]))
            table_vmem[pl.ds(16 * v, L)] = acc

        def body(idx_vmem, out_vmem):
            @pl.loop(0, BH)
            def _(bh):
                for tl in range(8):
                    for cc in range(8):
                        sl = pl.ds(16 * cc, L)
                        idxv = idx_vmem.at[0, bh, tl, sl][...]
                        wbase = idxv * V
                        for j in range(V):
                            vals = plsc.load_gather(table_vmem, [wbase + j])
                            out_vmem.at[j, 0, bh, tl, sl][...] = vals

        pltpu.emit_pipeline(
            body,
            grid=(TH, BHQ // BH),
            in_specs=[pl.BlockSpec((1, BH, 8, 128),
                                   index_map=lambda th, s: (th, s, 0, 0))],
            out_specs=[pl.BlockSpec((V, 1, BH, 8, 128),
                                    index_map=lambda th, s: (0, th, s, 0, 0))],
            core_axis_name=("c", "s"),
            dimension_semantics=(pltpu.PARALLEL, pltpu.PARALLEL),
        )(idx_hbm, out_hbm)

    out5 = sc_kern(idx4, weight)
    return out5.transpose(2, 4, 1, 3, 0).reshape(B, T, V)


# speedup vs baseline: 37.1278x; 3.8052x over previous
import dataclasses
import functools

import jax
import jax.numpy as jnp
from jax import lax
from jax.experimental import pallas as pl
from jax.experimental.pallas import tpu as pltpu
from jax.experimental.pallas import tpu_sc as plsc

B, T, V, C = 16384, 200, 10, 3
L = 16
TH = T // 8
BHQ = B // 128
BH = 4


def kernel(idx, weight):
    idx4 = idx.T.reshape(TH, 8, BHQ, 128).transpose(0, 2, 1, 3)
    mesh = plsc.VectorSubcoreMesh(core_axis_name="c", subcore_axis_name="s")

    @functools.partial(
        pl.kernel,
        out_type=jax.ShapeDtypeStruct((V, TH, BHQ, 8, 128), jnp.float32),
        mesh=mesh,
        scratch_types=[pltpu.VMEM((128,), jnp.float32),
                       pltpu.VMEM((V, C), jnp.float32)],
        compiler_params=dataclasses.replace(
            pltpu.CompilerParams(), needs_layout_passes=False),
    )
    def sc_kern(idx_hbm, w_hbm, out_hbm, table_vmem, w_vmem):
        pltpu.sync_copy(w_hbm, w_vmem)
        for v in range(7):
            e = lax.iota(jnp.int32, L) + (16 * v)
            k = jnp.minimum(e // V, V - 1)
            j2 = jnp.minimum(e - (e // V) * V, V - 1)
            acc = jnp.zeros((L,), jnp.float32)
            for c in range(C):
                cc = jnp.full((L,), c, jnp.int32)
                acc = acc + (plsc.load_gather(w_vmem, [k, cc]) *
                             plsc.load_gather(w_vmem, [j2, cc]))
            table_vmem[pl.ds(16 * v, L)] = acc

        def body(idx_vmem, out_vmem):
            @pl.loop(0, BH)
            def _(bh):
                for tl in range(8):
                    wbases = []
                    for cc in range(8):
                        idxv = idx_vmem.at[0, bh, tl, pl.ds(16 * cc, L)][...]
                        wbases.append(idxv * V)
                    for cc in range(8):
                        sl = pl.ds(16 * cc, L)
                        vals = [plsc.load_gather(table_vmem, [wbases[cc] + j])
                                for j in range(V)]
                        for j in range(V):
                            out_vmem.at[j, 0, bh, tl, sl][...] = vals[j]

        pltpu.emit_pipeline(
            body,
            grid=(TH, BHQ // BH),
            in_specs=[pl.BlockSpec((1, BH, 8, 128),
                                   index_map=lambda th, s: (th, s, 0, 0))],
            out_specs=[pl.BlockSpec((V, 1, BH, 8, 128),
                                    index_map=lambda th, s: (0, th, s, 0, 0))],
            core_axis_name=("c", "s"),
            dimension_semantics=(pltpu.PARALLEL, pltpu.PARALLEL),
        )(idx_hbm, out_hbm)

    out5 = sc_kern(idx4, weight)
    return out5.transpose(2, 4, 1, 3, 0).reshape(B, T, V)
